# speedup vs baseline: 1.6108x; 1.6108x over previous
import functools

import jax
import jax.numpy as jnp
from jax.experimental import pallas as pl
from jax.experimental.pallas import tpu as pltpu


def _silu(x):
    return x * jax.nn.sigmoid(x)


def _routing_weight(xt, wgate, e, TT, E):
    logits = jax.lax.dot_general(
        xt, wgate, (((1,), (1,)), ((), ())), preferred_element_type=jnp.float32
    )
    iota = jax.lax.broadcasted_iota(jnp.int32, (TT, E), 1)
    v0 = jnp.max(logits, axis=1, keepdims=True)
    e0 = jnp.min(jnp.where(logits == v0, iota, E), axis=1, keepdims=True)
    masked = jnp.where(iota == e0, -1e30, logits)
    v1 = jnp.max(masked, axis=1, keepdims=True)
    e1 = jnp.min(jnp.where(masked == v1, iota, E), axis=1, keepdims=True)
    r = jnp.exp(v1 - v0)
    p0 = 1.0 / (1.0 + r)
    p1 = r / (1.0 + r)
    return jnp.where(e0 == e, p0, 0.0) + jnp.where(e1 == e, p1, 0.0)


def _moe_body(x_ref, wgate_ref, wg_ref, wu_ref, wd_ref, out_ref, acc_ref, *, TT, E):
    e = pl.program_id(0)
    t = pl.program_id(1)
    xt = x_ref[...]
    w = _routing_weight(xt, wgate_ref[...], e, TT, E)

    xb = xt.astype(jnp.bfloat16)
    g = jax.lax.dot_general(
        xb, wg_ref[0].astype(jnp.bfloat16), (((1,), (1,)), ((), ())),
        preferred_element_type=jnp.float32)
    u = jax.lax.dot_general(
        xb, wu_ref[0].astype(jnp.bfloat16), (((1,), (1,)), ((), ())),
        preferred_element_type=jnp.float32)
    a = (_silu(g) * u).astype(jnp.bfloat16)
    y = jax.lax.dot_general(
        a, wd_ref[0].astype(jnp.bfloat16), (((1,), (1,)), ((), ())),
        preferred_element_type=jnp.float32)
    contrib = y * w

    sl = pl.ds(t * TT, TT)

    @pl.when(e == 0)
    def _():
        acc_ref[sl, :] = contrib

    @pl.when(e > 0)
    def _():
        acc_ref[sl, :] += contrib

    @pl.when(e == E - 1)
    def _():
        out_ref[...] = acc_ref[sl, :]


def _moe_dense(x2, W_gate, Wg, Wu, Wd, TT=256):
    T, C = x2.shape
    E, H, _ = Wg.shape
    nt = T // TT
    body = functools.partial(_moe_body, TT=TT, E=E)
    return pl.pallas_call(
        body,
        grid=(E, nt),
        in_specs=[
            pl.BlockSpec((TT, C), lambda e, t: (t, 0)),
            pl.BlockSpec((E, C), lambda e, t: (0, 0)),
            pl.BlockSpec((1, H, C), lambda e, t: (e, 0, 0)),
            pl.BlockSpec((1, H, C), lambda e, t: (e, 0, 0)),
            pl.BlockSpec((1, C, H), lambda e, t: (e, 0, 0)),
        ],
        out_specs=pl.BlockSpec((TT, C), lambda e, t: (t, 0)),
        out_shape=jax.ShapeDtypeStruct((T, C), jnp.float32),
        scratch_shapes=[pltpu.VMEM((T, C), jnp.float32)],
        compiler_params=pltpu.CompilerParams(
            dimension_semantics=("arbitrary", "arbitrary"),
        ),
    )(x2, W_gate, Wg, Wu, Wd)


def kernel(x, W_gate, Wg, Wu, Wd):
    B, T, C = x.shape
    out = _moe_dense(x.reshape(T, C), W_gate, Wg, Wu, Wd)
    return out.reshape(B, T, C)


# speedup vs baseline: 2.1574x; 1.3393x over previous
import functools

import jax
import jax.numpy as jnp
from jax import lax
from jax.experimental import pallas as pl
from jax.experimental.pallas import tpu as pltpu
from jax.experimental.pallas import tpu_sc as plsc

T = 2048
C = 768
E = 8
H = 1536
M = T * 2
BT = 256
NV = M // BT + E - 1
NW = 32
CHUNK = T // NW


def _route_body(x_ref, wgate_ref, s0_ref, s1_ref, p0_ref, p1_ref, cnt_ref):
    xt = x_ref[...]
    logits = lax.dot_general(
        xt, wgate_ref[...], (((1,), (1,)), ((), ())),
        preferred_element_type=jnp.float32)
    iota_e = lax.broadcasted_iota(jnp.int32, (T, E), 1)
    v0 = jnp.max(logits, axis=1, keepdims=True)
    e0 = jnp.min(jnp.where(logits == v0, iota_e, E), axis=1, keepdims=True)
    masked = jnp.where(iota_e == e0, -1e30, logits)
    v1 = jnp.max(masked, axis=1, keepdims=True)
    e1 = jnp.min(jnp.where(masked == v1, iota_e, E), axis=1, keepdims=True)
    r = jnp.exp(v1 - v0)
    p0_ref[...] = jnp.broadcast_to(1.0 / (1.0 + r), (T, 16))
    p1_ref[...] = jnp.broadcast_to(r / (1.0 + r), (T, 16))

    one0 = (iota_e == e0)
    one1 = (iota_e == e1)
    o01 = jnp.concatenate(
        [one0.astype(jnp.bfloat16), one1.astype(jnp.bfloat16)], axis=1)
    row_i = lax.broadcasted_iota(jnp.int32, (T, T), 0)
    col_i = lax.broadcasted_iota(jnp.int32, (T, T), 1)
    ls = (row_i > col_i).astype(jnp.bfloat16)
    r01 = lax.dot_general(
        ls, o01, (((1,), (0,)), ((), ())),
        preferred_element_type=jnp.float32)
    r0 = r01[:, :E]
    r1 = r01[:, E:]

    o0f = one0.astype(jnp.float32)
    o1f = one1.astype(jnp.float32)
    counts0 = jnp.sum(o0f, axis=0, keepdims=True)
    counts1 = jnp.sum(o1f, axis=0, keepdims=True)
    counts = counts0 + counts1
    lt8 = (lax.broadcasted_iota(jnp.int32, (E, E), 0)
           < lax.broadcasted_iota(jnp.int32, (E, E), 1)).astype(jnp.bfloat16)
    m01 = (one0.astype(jnp.bfloat16) + one1.astype(jnp.bfloat16))
    pref = lax.dot_general(
        m01, lt8, (((1,), (0,)), ((), ())),
        preferred_element_type=jnp.float32)
    offs = jnp.sum(pref, axis=0, keepdims=True)
    slot0 = jnp.sum(o0f * (offs + r0), axis=1, keepdims=True)
    slot1 = jnp.sum(o1f * (offs + counts0 + r1), axis=1, keepdims=True)
    s0_ref[...] = slot0.astype(jnp.int32)
    s1_ref[...] = slot1.astype(jnp.int32)
    cnt_ref[...] = counts.astype(jnp.int32)


def _route(x2, W_gate):
    return pl.pallas_call(
        _route_body,
        out_shape=(
            jax.ShapeDtypeStruct((T, 1), jnp.int32),
            jax.ShapeDtypeStruct((T, 1), jnp.int32),
            jax.ShapeDtypeStruct((T, 16), jnp.float32),
            jax.ShapeDtypeStruct((T, 16), jnp.float32),
            jax.ShapeDtypeStruct((1, E), jnp.int32),
        ),
    )(x2, W_gate)


def _dispatch_body(x_hbm, s0_hbm, s1_hbm, xs_hbm, idx_v, rows_v, sem):
    wid = lax.axis_index("s") * 2 + lax.axis_index("c")
    base = wid * CHUNK
    pltpu.sync_copy(x_hbm.at[pl.ds(base, CHUNK)], rows_v)
    pltpu.sync_copy(s0_hbm.at[pl.ds(base, CHUNK)], idx_v)
    pltpu.async_copy(rows_v, xs_hbm.at[idx_v], sem).wait()
    pltpu.sync_copy(s1_hbm.at[pl.ds(base, CHUNK)], idx_v)
    pltpu.async_copy(rows_v, xs_hbm.at[idx_v], sem).wait()


def _dispatch(x2, slot0, slot1):
    mesh = plsc.VectorSubcoreMesh(core_axis_name="c", subcore_axis_name="s")
    f = pl.kernel(
        _dispatch_body,
        mesh=mesh,
        out_type=jax.ShapeDtypeStruct((M, C), jnp.float32),
        scratch_types=[
            pltpu.VMEM((CHUNK,), jnp.int32),
            pltpu.VMEM((CHUNK, C), jnp.float32),
            pltpu.SemaphoreType.DMA,
        ],
    )
    return f(x2, slot0, slot1)


def _gmm_body(tid_ref, gid_ref, gs_ref, ge_ref,
              xs_ref, wg_ref, wu_ref, wd_ref, out_ref):
    v = pl.program_id(0)
    tile = tid_ref[v]
    rows = tile * BT + lax.broadcasted_iota(jnp.int32, (BT, 1), 0)
    active = (rows >= gs_ref[v]) & (rows < ge_ref[v])

    xb = xs_ref[...].astype(jnp.bfloat16)
    g = lax.dot_general(
        xb, wg_ref[0].astype(jnp.bfloat16), (((1,), (1,)), ((), ())),
        preferred_element_type=jnp.float32)
    u = lax.dot_general(
        xb, wu_ref[0].astype(jnp.bfloat16), (((1,), (1,)), ((), ())),
        preferred_element_type=jnp.float32)
    a = (g * jax.nn.sigmoid(g) * u).astype(jnp.bfloat16)
    y = lax.dot_general(
        a, wd_ref[0].astype(jnp.bfloat16), (((1,), (1,)), ((), ())),
        preferred_element_type=jnp.float32)
    yw = jnp.where(active, y, 0.0)

    prev = tid_ref[jnp.maximum(v - 1, 0)]
    first = (v == 0) | (prev != tile)

    @pl.when(first)
    def _():
        out_ref[...] = yw

    @pl.when(jnp.logical_not(first))
    def _():
        out_ref[...] += yw


def _gmm(xs, Wg, Wu, Wd, tile_ids, group_ids, group_start, group_end):
    grid_spec = pltpu.PrefetchScalarGridSpec(
        num_scalar_prefetch=4,
        grid=(NV,),
        in_specs=[
            pl.BlockSpec((BT, C), lambda v, tid, gid, gs, ge: (tid[v], 0)),
            pl.BlockSpec((1, H, C), lambda v, tid, gid, gs, ge: (gid[v], 0, 0)),
            pl.BlockSpec((1, H, C), lambda v, tid, gid, gs, ge: (gid[v], 0, 0)),
            pl.BlockSpec((1, C, H), lambda v, tid, gid, gs, ge: (gid[v], 0, 0)),
        ],
        out_specs=pl.BlockSpec((BT, C), lambda v, tid, gid, gs, ge: (tid[v], 0)),
    )
    return pl.pallas_call(
        _gmm_body,
        grid_spec=grid_spec,
        out_shape=jax.ShapeDtypeStruct((M, C), jnp.float32),
        compiler_params=pltpu.CompilerParams(
            dimension_semantics=("arbitrary",),
        ),
    )(tile_ids, group_ids, group_start, group_end, xs, Wg, Wu, Wd)


def _combine_body(ys_hbm, s0_hbm, s1_hbm, p0_hbm, p1_hbm, out_hbm,
                  idx_v, p0_v, p1_v, buf0, buf1, sem):
    wid = lax.axis_index("s") * 2 + lax.axis_index("c")
    base = wid * CHUNK
    pltpu.sync_copy(s0_hbm.at[pl.ds(base, CHUNK)], idx_v)
    pltpu.async_copy(ys_hbm.at[idx_v], buf0, sem).wait()
    pltpu.sync_copy(s1_hbm.at[pl.ds(base, CHUNK)], idx_v)
    pltpu.async_copy(ys_hbm.at[idx_v], buf1, sem).wait()
    pltpu.sync_copy(p0_hbm.at[pl.ds(base, CHUNK)], p0_v)
    pltpu.sync_copy(p1_hbm.at[pl.ds(base, CHUNK)], p1_v)

    def row(rr, carry):
        s0 = p0_v[rr, :]
        s1 = p1_v[rr, :]
        for cc in range(C // 16):
            sl = pl.ds(cc * 16, 16)
            buf0[rr, sl] = buf0[rr, sl] * s0 + buf1[rr, sl] * s1
        return carry

    lax.fori_loop(0, CHUNK, row, 0)
    pltpu.sync_copy(buf0, out_hbm.at[pl.ds(base, CHUNK)])


def _combine(ys, slot0, slot1, p0, p1):
    mesh = plsc.VectorSubcoreMesh(core_axis_name="c", subcore_axis_name="s")
    f = pl.kernel(
        _combine_body,
        mesh=mesh,
        out_type=jax.ShapeDtypeStruct((T, C), jnp.float32),
        scratch_types=[
            pltpu.VMEM((CHUNK,), jnp.int32),
            pltpu.VMEM((CHUNK, 16), jnp.float32),
            pltpu.VMEM((CHUNK, 16), jnp.float32),
            pltpu.VMEM((CHUNK, C), jnp.float32),
            pltpu.VMEM((CHUNK, C), jnp.float32),
            pltpu.SemaphoreType.DMA,
        ],
    )
    return f(ys, slot0, slot1, p0, p1)


def _group_metadata(counts):
    counts = counts.astype(jnp.int32)
    ge_ = jnp.cumsum(counts)
    go_ = ge_ - counts
    first_tile = go_ // BT
    last_tile = jnp.where(counts > 0, (ge_ + BT - 1) // BT - 1, first_tile)
    ntiles = jnp.where(counts > 0, last_tile - first_tile + 1, 0)
    cum = jnp.cumsum(ntiles)
    vstart = cum - ntiles
    nreal = cum[-1]
    v = jnp.arange(NV, dtype=jnp.int32)
    vc = jnp.minimum(v, nreal - 1)
    e_of_v = jnp.searchsorted(cum, vc, side="right").astype(jnp.int32)
    tile_ids = first_tile[e_of_v] + (vc - vstart[e_of_v])
    valid = v < nreal
    group_start = jnp.where(valid, go_[e_of_v], 0)
    group_end = jnp.where(valid, ge_[e_of_v], 0)
    return (tile_ids.astype(jnp.int32), e_of_v,
            group_start.astype(jnp.int32), group_end.astype(jnp.int32))


def kernel(x, W_gate, Wg, Wu, Wd):
    B = x.shape[0]
    x2 = x.reshape(T, C)
    s0, s1, p0, p1, counts = _route(x2, W_gate)
    s0 = s0.reshape(T)
    s1 = s1.reshape(T)
    tile_ids, group_ids, group_start, group_end = _group_metadata(
        counts.reshape(E))
    xs = _dispatch(x2, s0, s1)
    ys = _gmm(xs, Wg, Wu, Wd, tile_ids, group_ids, group_start, group_end)
    out = _combine(ys, s0, s1, p0, p1)
    return out.reshape(B, T, C)
